# baseline (speedup 1.0000x reference)
import functools

import jax
import jax.numpy as jnp
from jax import lax
from jax.experimental import pallas as pl
from jax.experimental.pallas import tpu as pltpu
from jax.experimental.pallas import tpu_sc as plsc

N = 4096
E = 1048576
NC = 2
NS = 16
NW = NC * NS
LANES = 16
EPW = E // NW
CHUNK = 8192
NCHUNK = EPW // CHUNK
TIERS = (2048, 8192, EPW)
ROUNDS = 8

_mesh = plsc.VectorSubcoreMesh(core_axis_name="c", subcore_axis_name="s")
_params = pltpu.CompilerParams(needs_layout_passes=False)

_LIST_SCRATCH = [
    pltpu.VMEM((EPW,), jnp.int32),
    pltpu.VMEM((EPW,), jnp.float32),
    pltpu.VMEM((LANES,), jnp.int32),
    pltpu.SemaphoreType.DMA,
]


def _splat(x):
    return jnp.full((LANES,), x, dtype=jnp.int32)


def _pad_cyclic(idx_buf, w_buf, cnt, total):
    iota = lax.iota(jnp.int32, LANES)

    def body(t, carry):
        p = _splat(cnt + t * LANES) + iota
        pm = p - (p // _splat(cnt)) * _splat(cnt)
        src = plsc.load_gather(idx_buf, [pm])
        srw = plsc.load_gather(w_buf, [pm])
        m = p < _splat(total)
        plsc.store_scatter(idx_buf, [p], src, mask=m)
        plsc.store_scatter(w_buf, [p], srw, mask=m)
        return carry

    npad = (total - cnt + LANES - 1) // LANES
    lax.fori_loop(0, npad, body, 0)


def _tier_bounds(t):
    i = TIERS.index(t)
    return 0 if i == 0 else TIERS[i - 1]


def _tier_scatter(out_hbm, idx_buf, w_buf, sem, cnt):
    for t in TIERS:
        lo = _tier_bounds(t)

        @pl.when(jnp.logical_and(cnt > lo, cnt <= t))
        def _():
            pltpu.async_copy(
                w_buf.at[pl.ds(0, t)],
                out_hbm.at[idx_buf.at[pl.ds(0, t)]], sem).wait()


def _tier_gather(out_hbm, idx_buf, gbuf, sem, cnt):
    for t in TIERS:
        lo = _tier_bounds(t)

        @pl.when(jnp.logical_and(cnt > lo, cnt <= t))
        def _():
            pltpu.async_copy(
                out_hbm.at[idx_buf.at[pl.ds(0, t)]],
                gbuf.at[pl.ds(0, t)], sem).wait()


def _tier_of(cnt):
    t = jnp.int32(TIERS[-1])
    for tt in reversed(TIERS[:-1]):
        t = jnp.where(cnt <= tt, jnp.int32(tt), t)
    return t


@functools.partial(
    pl.kernel,
    mesh=_mesh,
    scratch_types=_LIST_SCRATCH + [
        pltpu.VMEM((CHUNK,), jnp.int32),
        pltpu.VMEM((CHUNK,), jnp.int32),
    ],
    compiler_params=_params,
)
def _phase_a(w_hbm, i_hbm, j_hbm, out_hbm, idx_hbm, wl_hbm, cnt_hbm,
             idx_buf, w_buf, cntbuf, sem, st_i, st_j):
    c = lax.axis_index("c")
    s = lax.axis_index("s")
    wid = s * NC + c
    c12 = _splat(12)
    base = wid * EPW

    pltpu.sync_copy(w_hbm.at[pl.ds(base, EPW)], w_buf)

    def fchunk(ch, carry):
        pltpu.sync_copy(i_hbm.at[pl.ds(base + ch * CHUNK, CHUNK)], st_i)
        pltpu.sync_copy(j_hbm.at[pl.ds(base + ch * CHUNK, CHUNK)], st_j)

        def fvec(v, carry):
            iv = st_i[pl.ds(v * LANES, LANES)]
            jv = st_j[pl.ds(v * LANES, LANES)]
            flat = jnp.bitwise_or(lax.shift_left(iv, c12), jv)
            idx_buf[pl.ds(ch * CHUNK + v * LANES, LANES)] = flat
            return carry

        return lax.fori_loop(0, CHUNK // LANES, fvec, carry, unroll=8)

    lax.fori_loop(0, NCHUNK, fchunk, 0)

    pltpu.async_copy(w_buf, out_hbm.at[idx_buf], sem).wait()

    cntbuf[...] = _splat(EPW)
    pltpu.sync_copy(cntbuf, cnt_hbm.at[wid])
    pltpu.sync_copy(idx_buf, idx_hbm.at[wid])
    pltpu.sync_copy(w_buf, wl_hbm.at[wid])


@functools.partial(
    pl.kernel,
    mesh=_mesh,
    scratch_types=_LIST_SCRATCH + [
        pltpu.VMEM((EPW,), jnp.float32),
    ],
    compiler_params=_params,
)
def _phase_b(out_hbm, idx_hbm, wl_hbm, cnt_hbm,
             idx_buf, w_buf, cntbuf, sem, gbuf):
    c = lax.axis_index("c")
    s = lax.axis_index("s")
    wid = s * NC + c
    iota = lax.iota(jnp.int32, LANES)

    pltpu.sync_copy(cnt_hbm.at[wid], cntbuf)
    cn = jnp.max(cntbuf[...])

    @pl.when(cn > 0)
    def _():
        pltpu.sync_copy(idx_hbm.at[wid], idx_buf)
        pltpu.sync_copy(wl_hbm.at[wid], w_buf)
        _tier_gather(out_hbm, idx_buf, gbuf, sem, cn)
        cnv = _splat(cn)

        def cvec(v, wc):
            sl = pl.ds(v * LANES, LANES)
            gath = gbuf[sl]
            myw = w_buf[sl]
            myidx = idx_buf[sl]
            qv = _splat(v * LANES) + iota
            alive = jnp.logical_and(gath < myw, qv < cnv)
            plsc.store_compressed(idx_buf.at[pl.ds(wc, LANES)], myidx,
                                  mask=alive)
            plsc.store_compressed(w_buf.at[pl.ds(wc, LANES)], myw,
                                  mask=alive)
            return wc + plsc.all_reduce_population_count(alive)[0]

        nv = lax.shift_right_arithmetic(cn + LANES - 1, 4)
        cnt2 = lax.fori_loop(0, nv, cvec, jnp.int32(0))

        @pl.when(cnt2 > 0)
        def _():
            _pad_cyclic(idx_buf, w_buf, cnt2, _tier_of(cnt2))
            _tier_scatter(out_hbm, idx_buf, w_buf, sem, cnt2)
            pltpu.sync_copy(idx_buf, idx_hbm.at[wid])
            pltpu.sync_copy(w_buf, wl_hbm.at[wid])

        cntbuf[...] = _splat(cnt2)
        pltpu.sync_copy(cntbuf, cnt_hbm.at[wid])


def kernel(weights, edge_i, edge_j):
    out = jax.new_ref(jnp.zeros((N * N,), jnp.float32))
    idx_l = jax.new_ref(jnp.zeros((NW, EPW), jnp.int32))
    w_l = jax.new_ref(jnp.zeros((NW, EPW), jnp.float32))
    cnt_l = jax.new_ref(jnp.zeros((NW, LANES), jnp.int32))
    _phase_a(weights, edge_i, edge_j, out, idx_l, w_l, cnt_l)
    for _ in range(ROUNDS):
        _phase_b(out, idx_l, w_l, cnt_l)
    return out[...].reshape(N, N)
